# baseline (speedup 1.0000x reference)
import functools

import jax
import jax.numpy as jnp
from jax import lax
from jax.experimental import pallas as pl
from jax.experimental.pallas import tpu as pltpu
from jax.experimental.pallas import tpu_sc as plsc

VOCAB = 1000001
EMBED_DIM = 64
BATCH = 4096
SEQ_LEN = 50
VPAD = 1000008
NBLK = VOCAB // 128
TAIL = VOCAB - NBLK * 128
NC = 2
NS = 16
NW = NC * NS
BLK_W = -(-NBLK // NW)
BROWS_W = BATCH // NW

_mesh = plsc.VectorSubcoreMesh(core_axis_name="c", subcore_axis_name="s")
_params = pltpu.CompilerParams(use_tc_tiling_on_sc=True,
                               needs_layout_passes=False)


TBLK = 512
NTBLK = -(-VOCAB // TBLK)
VOUT = NTBLK * TBLK


def _tT_body(x_ref, o_ref):
    xt = x_ref[...].T
    o_ref[...] = jnp.concatenate(
        [xt, jnp.zeros((TBLK, 128 - EMBED_DIM), jnp.float32)], axis=1)


_tc_transpose = pl.pallas_call(
    _tT_body,
    grid=(NTBLK,),
    in_specs=[pl.BlockSpec((EMBED_DIM, TBLK), lambda j: (0, j))],
    out_specs=pl.BlockSpec((TBLK, 128), lambda j: (j, 0)),
    out_shape=jax.ShapeDtypeStruct((VOUT, 128), jnp.float32),
)


@functools.partial(
    pl.kernel,
    mesh=_mesh,
    out_type=jax.ShapeDtypeStruct((BATCH, SEQ_LEN, 128), jnp.float32),
    scratch_types=[
        pltpu.VMEM((SEQ_LEN, BROWS_W), jnp.int32),
        pltpu.VMEM((BROWS_W, SEQ_LEN), jnp.int32),
        pltpu.VMEM((SEQ_LEN, 128), jnp.float32),
        pltpu.VMEM((SEQ_LEN, 128), jnp.float32),
        pltpu.SemaphoreType.DMA,
        pltpu.SemaphoreType.DMA,
        pltpu.SemaphoreType.DMA,
        pltpu.SemaphoreType.DMA,
    ],
    compiler_params=_params,
)
def _sc_gather(idsT_hbm, tab_hbm, out_hbm, idx_v, idxT_v, buf0, buf1,
               gs0, gs1, os0, os1):
    wid = lax.axis_index("s") * NC + lax.axis_index("c")
    b0 = wid * BROWS_W
    bufs = (buf0, buf1)
    gsems = (gs0, gs1)
    osems = (os0, os1)
    iota = lax.iota(jnp.int32, 16)
    rows = [ib * 16 + iota for ib in range(8)]

    pltpu.sync_copy(
        idsT_hbm.at[:, pl.ds(pl.multiple_of(b0, 128), BROWS_W)], idx_v)
    for s in range(SEQ_LEN):
        col = jnp.full((16,), s, jnp.int32)
        for ib in range(8):
            v = idx_v[s, pl.ds(ib * 16, 16)]
            plsc.store_scatter(idxT_v, [rows[ib], col], v)

    def fire(jj, k):
        pltpu.async_copy(tab_hbm.at[idxT_v.at[jj]], bufs[k], gsems[k])

    def drain_gather(jj, k):
        pltpu.make_async_copy(
            tab_hbm.at[idxT_v.at[jj]], bufs[k], gsems[k]).wait()

    def wait_out(k):
        pltpu.make_async_copy(
            bufs[k], out_hbm.at[0], osems[k]).wait()

    fire(0, 0)

    def step(g, carry):
        for kk in range(2):
            jj = g * 2 + kk
            k = kk
            nk = 1 - kk

            @pl.when(jj + 1 < BROWS_W)
            def _():
                fire(jj + 1, nk)
            drain_gather(jj, k)
            @pl.when(jj >= 2)
            def _():
                wait_out(k)
            pltpu.async_copy(bufs[k], out_hbm.at[b0 + jj], osems[k])
        return carry
    lax.fori_loop(0, BROWS_W // 2, step, 0)
    wait_out(0)
    wait_out(1)


def kernel(input_ids, table):
    tT = table.T
    scratch = _tc_transpose(tT)
    idsT = input_ids.T.astype(jnp.int32)
    out = _sc_gather(idsT, scratch)
    return out[:, :, :EMBED_DIM]


# speedup vs baseline: 1.7395x; 1.7395x over previous
import functools

import jax
import jax.numpy as jnp
from jax import lax
from jax.experimental import pallas as pl
from jax.experimental.pallas import tpu as pltpu
from jax.experimental.pallas import tpu_sc as plsc

EMBED_DIM = 64
BATCH = 4096
SEQ_LEN = 50
NC = 2
NS = 16
NW = NC * NS
ROWS_W = BATCH // NW
GB = 16
NCH = ROWS_W // GB

_mesh = plsc.VectorSubcoreMesh(core_axis_name="c", subcore_axis_name="s")


@functools.partial(
    pl.kernel,
    mesh=_mesh,
    out_type=jax.ShapeDtypeStruct((BATCH, SEQ_LEN, EMBED_DIM), jnp.float32),
    scratch_types=[
        pltpu.VMEM((ROWS_W, SEQ_LEN), jnp.int32),
        pltpu.VMEM((GB, SEQ_LEN, EMBED_DIM), jnp.float32),
        pltpu.VMEM((GB, SEQ_LEN, EMBED_DIM), jnp.float32),
        pltpu.SemaphoreType.DMA,
        pltpu.SemaphoreType.DMA,
        pltpu.SemaphoreType.DMA,
        pltpu.SemaphoreType.DMA,
    ],
    compiler_params=pltpu.CompilerParams(use_tc_tiling_on_sc=False),
)
def _sc_gather(ids_hbm, table_hbm, out_hbm, idx_v, buf0, buf1, gs0, gs1, os0, os1):
    wid = lax.axis_index("s") * NC + lax.axis_index("c")
    rbase = wid * ROWS_W
    pltpu.sync_copy(ids_hbm.at[pl.ds(rbase, ROWS_W)], idx_v)

    bufs = (buf0, buf1)
    gsems = (gs0, gs1)
    osems = (os0, os1)
    gathers = [None] * NCH
    outs = [None] * NCH

    def start_chunk(j, b):
        return [
            pltpu.async_copy(
                table_hbm.at[idx_v.at[j * GB + i]], bufs[b].at[i], gsems[b])
            for i in range(GB)
        ]

    gathers[0] = start_chunk(0, 0)
    for j in range(NCH):
        b = j & 1
        nb = (j + 1) & 1
        if j + 1 < NCH:
            if j >= 1:
                outs[j - 1].wait()
            gathers[j + 1] = start_chunk(j + 1, nb)
        for h in gathers[j]:
            h.wait()
        outs[j] = pltpu.async_copy(
            bufs[b], out_hbm.at[pl.ds(rbase + j * GB, GB)], osems[b])
    outs[NCH - 2].wait()
    outs[NCH - 1].wait()


def kernel(input_ids, table):
    return _sc_gather(input_ids.astype(jnp.int32), table)
